# Initial kernel scaffold; baseline (speedup 1.0000x reference)
import jax
import jax.numpy as jnp
from jax.experimental import pallas as pl


def kernel(x, emb_table, W, b):
    raise NotImplementedError("write your pallas kernel here")


# speedup vs baseline: 11.8099x; 11.8099x over previous
import functools

import jax
import jax.numpy as jnp
from jax import lax
from jax.experimental import pallas as pl
from jax.experimental.pallas import tpu as pltpu
from jax.experimental.pallas import tpu_sc as plsc

_NC, _NS, _LANES = 2, 16, 16
_NW = _NC * _NS


def _sc_partial_sums(xT, emb_table):
    L, B = xT.shape
    _, DIM = emb_table.shape
    bpw = B // _NW

    mesh = plsc.VectorSubcoreMesh(core_axis_name="c", subcore_axis_name="s")

    @functools.partial(
        pl.kernel,
        out_type=jax.ShapeDtypeStruct((_NW, L, DIM), jnp.float32),
        mesh=mesh,
        scratch_types=[
            pltpu.VMEM((bpw,), jnp.int32),
            pltpu.VMEM((bpw, DIM), jnp.float32),
            pltpu.VMEM((L, DIM), jnp.float32),
            pltpu.SemaphoreType.DMA,
        ],
        compiler_params=pltpu.CompilerParams(use_tc_tiling_on_sc=False),
    )
    def k(xT_hbm, table_hbm, parts_hbm, idx_v, rows_v, part_v, sem):
        wid = lax.axis_index("s") * _NC + lax.axis_index("c")
        base = wid * bpw

        def body_l(l, carry):
            pltpu.sync_copy(xT_hbm.at[l, pl.ds(base, bpw)], idx_v)
            pltpu.async_copy(table_hbm.at[idx_v], rows_v, sem).wait()

            def body_g(g, acc):
                a0, a1 = acc
                return (a0 + rows_v[g, pl.ds(0, _LANES)],
                        a1 + rows_v[g, pl.ds(_LANES, _LANES)])

            z = jnp.zeros((_LANES,), jnp.float32)
            a0, a1 = lax.fori_loop(0, bpw, body_g, (z, z), unroll=8)
            part_v[l, pl.ds(0, _LANES)] = a0
            part_v[l, pl.ds(_LANES, _LANES)] = a1
            return carry

        lax.fori_loop(0, L, body_l, 0)
        pltpu.sync_copy(part_v, parts_hbm.at[wid])

    return k(xT, emb_table)


def _tc_finish(parts, W, b2d, n_total):
    def body(parts_ref, w_ref, b_ref, out_ref):
        s = jnp.sum(parts_ref[...], axis=0) * (1.0 / n_total)
        out_ref[...] = lax.dot_general(
            s, w_ref[...], (((1,), (1,)), ((), ())),
            preferred_element_type=jnp.float32) + b_ref[...]

    L, DIM = parts.shape[1], parts.shape[2]
    return pl.pallas_call(
        body,
        out_shape=jax.ShapeDtypeStruct((L, DIM), jnp.float32),
    )(parts, W, b2d)


def kernel(x, emb_table, W, b):
    x = x.astype(jnp.int32)
    B, L = x.shape
    xT = x.T
    parts = _sc_partial_sums(xT, emb_table)
    return _tc_finish(parts, W, b.reshape(1, -1), B)


# speedup vs baseline: 15.8587x; 1.3428x over previous
import functools

import jax
import jax.numpy as jnp
from jax import lax
from jax.experimental import pallas as pl
from jax.experimental.pallas import tpu as pltpu
from jax.experimental.pallas import tpu_sc as plsc

_NC, _NS, _LANES = 2, 16, 16
_NW = _NC * _NS
_CH = 512


def _sc_col_sums(xT, emb_table):
    L, B = xT.shape
    _, DIM = emb_table.shape
    nch = B // _CH
    n_iter = (L + _NW - 1) // _NW

    mesh = plsc.VectorSubcoreMesh(core_axis_name="c", subcore_axis_name="s")

    @functools.partial(
        pl.kernel,
        out_type=jax.ShapeDtypeStruct((L, DIM), jnp.float32),
        mesh=mesh,
        scratch_types=[
            pltpu.VMEM((B,), jnp.int32),
            pltpu.VMEM((_CH, DIM), jnp.float32),
            pltpu.VMEM((_CH, DIM), jnp.float32),
            pltpu.VMEM((DIM,), jnp.float32),
            pltpu.SemaphoreType.DMA,
            pltpu.SemaphoreType.DMA,
        ],
        compiler_params=pltpu.CompilerParams(use_tc_tiling_on_sc=False),
    )
    def k(xT_hbm, table_hbm, out_hbm, idx_v, acc_a, acc_b, row_v, sem_a, sem_b):
        wid = lax.axis_index("s") * _NC + lax.axis_index("c")

        def body_i(i, carry):
            l = i * _NW + wid

            @pl.when(l < L)
            def _():
                pltpu.sync_copy(xT_hbm.at[l], idx_v)
                pltpu.async_copy(
                    table_hbm.at[idx_v.at[pl.ds(0, _CH)]], acc_a, sem_a)
                pltpu.async_copy(
                    table_hbm.at[idx_v.at[pl.ds(_CH, _CH)]], acc_b, sem_b)

                def pair(p, c2):
                    pltpu.make_async_copy(
                        table_hbm.at[idx_v.at[pl.ds(0, _CH)]], acc_a, sem_a
                    ).wait()
                    pltpu.async_copy(
                        table_hbm.at[idx_v.at[pl.ds(c2 * _CH, _CH)]],
                        acc_a, sem_a, add=True)
                    pltpu.make_async_copy(
                        table_hbm.at[idx_v.at[pl.ds(0, _CH)]], acc_b, sem_b
                    ).wait()
                    pltpu.async_copy(
                        table_hbm.at[idx_v.at[pl.ds((c2 + 1) * _CH, _CH)]],
                        acc_b, sem_b, add=True)
                    return c2 + 2

                lax.fori_loop(1, nch // 2, pair, 2)
                pltpu.make_async_copy(
                    table_hbm.at[idx_v.at[pl.ds(0, _CH)]], acc_a, sem_a).wait()
                pltpu.make_async_copy(
                    table_hbm.at[idx_v.at[pl.ds(0, _CH)]], acc_b, sem_b).wait()

                def red(g, acc):
                    a0, a1 = acc
                    a0 = a0 + acc_a[g, pl.ds(0, _LANES)] + acc_b[g, pl.ds(0, _LANES)]
                    a1 = a1 + acc_a[g, pl.ds(_LANES, _LANES)] + acc_b[g, pl.ds(_LANES, _LANES)]
                    return (a0, a1)

                z = jnp.zeros((_LANES,), jnp.float32)
                a0, a1 = lax.fori_loop(0, _CH, red, (z, z), unroll=8)
                row_v[pl.ds(0, _LANES)] = a0
                row_v[pl.ds(_LANES, _LANES)] = a1
                pltpu.sync_copy(row_v, out_hbm.at[l])

            return carry

        lax.fori_loop(0, n_iter, body_i, 0)

    return k(xT, emb_table)


def _tc_finish(sums, W, b2d, n_total):
    def body(sums_ref, w_ref, b_ref, out_ref):
        m = sums_ref[...] * (1.0 / n_total)
        out_ref[...] = lax.dot_general(
            m, w_ref[...], (((1,), (1,)), ((), ())),
            preferred_element_type=jnp.float32) + b_ref[...]

    L, DIM = sums.shape
    return pl.pallas_call(
        body,
        out_shape=jax.ShapeDtypeStruct((L, DIM), jnp.float32),
    )(sums, W, b2d)


def kernel(x, emb_table, W, b):
    x = x.astype(jnp.int32)
    B, L = x.shape
    xT = x.T
    sums = _sc_col_sums(xT, emb_table)
    return _tc_finish(sums, W, b.reshape(1, -1), B)
